# Initial kernel scaffold; baseline (speedup 1.0000x reference)
import functools

import jax
import jax.numpy as jnp
from jax import lax
from jax.experimental import pallas as pl
from jax.experimental.pallas import tpu as pltpu
from jax.experimental.pallas import tpu_sc as plsc

_N_NODES = 10000
_D = 128
_N_EDGES = 320000
_NUM_HOPS = 3

_NW = 32
_CHUNK = 128
_NCHUNK = 79
_EDGES_PAD = _NW * _NCHUNK * _CHUNK
_N_PAD = 10240
_ROWS_PER_SUB = _N_PAD // 16


def _hop_body(h_hbm, row_hbm, col_hbm, out_hbm, colv, rowv, rows_v, wb_v, acc, sem):
    c = lax.axis_index("c")
    s = lax.axis_index("s")
    wid = s * 2 + c

    pltpu.sync_copy(col_hbm.at[wid], colv)
    pltpu.sync_copy(row_hbm.at[wid], rowv)

    zero = jnp.zeros((16,), jnp.float32)

    def _zrow(i, carry):
        for q in range(_D // 16):
            wb_v[i, pl.ds(q * 16, 16)] = zero
        return carry

    lax.fori_loop(0, _ROWS_PER_SUB, _zrow, 0)
    pltpu.sync_copy(wb_v, acc.at[pl.ds(s * _ROWS_PER_SUB, _ROWS_PER_SUB)])
    plsc.subcore_barrier()

    def _edge(j, carry):
        pltpu.async_copy(h_hbm.at[colv.at[j]], rows_v, sem).wait()
        pltpu.sync_copy(rows_v, acc.at[rowv.at[j]], add=True)
        return carry

    lax.fori_loop(0, _NCHUNK, _edge, 0)
    plsc.subcore_barrier()

    base = s * _ROWS_PER_SUB
    pltpu.sync_copy(acc.at[pl.ds(base, _ROWS_PER_SUB)], wb_v)
    pltpu.sync_copy(wb_v, out_hbm.at[c, pl.ds(base, _ROWS_PER_SUB)])


@jax.jit
def _hop_sc(h, row3, col3):
    mesh = plsc.VectorSubcoreMesh(core_axis_name="c", subcore_axis_name="s")
    return pl.kernel(
        _hop_body,
        out_type=jax.ShapeDtypeStruct((2, _N_PAD, _D), jnp.float32),
        mesh=mesh,
        scratch_types=[
            pltpu.VMEM((_NCHUNK, _CHUNK), jnp.int32),
            pltpu.VMEM((_NCHUNK, _CHUNK), jnp.int32),
            pltpu.VMEM((_CHUNK, _D), jnp.float32),
            pltpu.VMEM((_ROWS_PER_SUB, _D), jnp.float32),
            pltpu.VMEM_SHARED((_N_PAD, _D), jnp.float32),
            pltpu.SemaphoreType.DMA,
        ],
    )(h, row3, col3)


def _norm2_body(p_ref, o_ref):
    su = p_ref[0] + p_ref[1]
    n = jnp.sqrt(jnp.sum(su * su, axis=1, keepdims=True))
    o_ref[...] = su / jnp.maximum(n, 1e-12)


@jax.jit
def _norm2(p):
    blk = 640
    return pl.pallas_call(
        _norm2_body,
        out_shape=jax.ShapeDtypeStruct((_N_PAD, _D), jnp.float32),
        grid=(_N_PAD // blk,),
        in_specs=[pl.BlockSpec((2, blk, _D), lambda i: (0, i, 0))],
        out_specs=pl.BlockSpec((blk, _D), lambda i: (i, 0)),
    )(p)


def _norm1_body(x_ref, o_ref):
    xb = x_ref[...]
    n = jnp.sqrt(jnp.sum(xb * xb, axis=1, keepdims=True))
    o_ref[...] = xb / jnp.maximum(n, 1e-12)


@jax.jit
def _norm1(x):
    blk = 640
    return pl.pallas_call(
        _norm1_body,
        out_shape=jax.ShapeDtypeStruct((_N_PAD, _D), jnp.float32),
        grid=(_N_PAD // blk,),
        in_specs=[pl.BlockSpec((blk, _D), lambda i: (i, 0))],
        out_specs=pl.BlockSpec((blk, _D), lambda i: (i, 0)),
    )(x)


def kernel(x, edge_index):
    row = edge_index[0].astype(jnp.int32)
    col = edge_index[1].astype(jnp.int32)
    npad = _EDGES_PAD - _N_EDGES
    rowp = jnp.concatenate([row, jnp.full((npad,), _N_NODES, jnp.int32)])
    colp = jnp.concatenate([col, jnp.zeros((npad,), jnp.int32)])
    row3 = rowp.reshape(_NW, _NCHUNK, _CHUNK)
    col3 = colp.reshape(_NW, _NCHUNK, _CHUNK)

    xpad = jnp.pad(x, ((0, _N_PAD - _N_NODES), (0, 0)))
    h = _norm1(xpad)
    outs = [h[:_N_NODES]]
    for _ in range(_NUM_HOPS):
        partials = _hop_sc(h, row3, col3)
        h = _norm2(partials)
        outs.append(h[:_N_NODES])
    return jnp.stack(outs)


# speedup vs baseline: 4.8628x; 4.8628x over previous
import functools

import jax
import jax.numpy as jnp
from jax import lax
from jax.experimental import pallas as pl
from jax.experimental.pallas import tpu as pltpu
from jax.experimental.pallas import tpu_sc as plsc

_N_NODES = 10000
_D = 128
_N_EDGES = 320000
_NUM_HOPS = 3

_NW = 32
_CHUNK = 128
_NCHUNK = 79
_EDGES_PAD = _NW * _NCHUNK * _CHUNK
_N_PAD = 10240
_ROWS_PER_SUB = _N_PAD // 16
_ROWS_WB = 64
_WB_ITERS = _ROWS_PER_SUB // _ROWS_WB


def _hop_body(h_hbm, row_hbm, col_hbm, out_hbm, colv, rowv, rows_v, wb_v, acc, sem):
    c = lax.axis_index("c")
    s = lax.axis_index("s")
    wid = s * 2 + c

    pltpu.sync_copy(col_hbm.at[wid], colv)
    pltpu.sync_copy(row_hbm.at[wid], rowv)

    zero = jnp.zeros((16,), jnp.float32)

    def _zrow(i, carry):
        for q in range(_D // 16):
            wb_v[i, pl.ds(q * 16, 16)] = zero
        return carry

    lax.fori_loop(0, _ROWS_WB, _zrow, 0)

    def _zacc(t, carry):
        pltpu.sync_copy(wb_v, acc.at[pl.ds(s * _ROWS_PER_SUB + t * _ROWS_WB, _ROWS_WB)])
        return carry

    lax.fori_loop(0, _WB_ITERS, _zacc, 0)
    plsc.subcore_barrier()

    def _edge(j, carry):
        pltpu.async_copy(h_hbm.at[colv.at[j]], rows_v, sem).wait()
        pltpu.sync_copy(rows_v, acc.at[rowv.at[j]], add=True)
        return carry

    lax.fori_loop(0, _NCHUNK, _edge, 0)
    plsc.subcore_barrier()

    base = s * _ROWS_PER_SUB

    def _wback(t, carry):
        pltpu.sync_copy(acc.at[pl.ds(base + t * _ROWS_WB, _ROWS_WB)], wb_v)
        pltpu.sync_copy(wb_v, out_hbm.at[c, pl.ds(base + t * _ROWS_WB, _ROWS_WB)])
        return carry

    lax.fori_loop(0, _WB_ITERS, _wback, 0)


@jax.jit
def _hop_sc(h, row3, col3):
    mesh = plsc.VectorSubcoreMesh(core_axis_name="c", subcore_axis_name="s")
    return pl.kernel(
        _hop_body,
        out_type=jax.ShapeDtypeStruct((2, _N_PAD, _D), jnp.float32),
        mesh=mesh,
        scratch_types=[
            pltpu.VMEM((_NCHUNK, _CHUNK), jnp.int32),
            pltpu.VMEM((_NCHUNK, _CHUNK), jnp.int32),
            pltpu.VMEM((_CHUNK, _D), jnp.float32),
            pltpu.VMEM((_ROWS_WB, _D), jnp.float32),
            pltpu.VMEM_SHARED((_N_PAD, _D), jnp.float32),
            pltpu.SemaphoreType.DMA,
        ],
    )(h, row3, col3)


def _norm2_body(p_ref, o_ref):
    su = p_ref[0] + p_ref[1]
    n = jnp.sqrt(jnp.sum(su * su, axis=1, keepdims=True))
    o_ref[...] = su / jnp.maximum(n, 1e-12)


@jax.jit
def _norm2(p):
    blk = 640
    return pl.pallas_call(
        _norm2_body,
        out_shape=jax.ShapeDtypeStruct((_N_PAD, _D), jnp.float32),
        grid=(_N_PAD // blk,),
        in_specs=[pl.BlockSpec((2, blk, _D), lambda i: (0, i, 0))],
        out_specs=pl.BlockSpec((blk, _D), lambda i: (i, 0)),
    )(p)


def _norm1_body(x_ref, o_ref):
    xb = x_ref[...]
    n = jnp.sqrt(jnp.sum(xb * xb, axis=1, keepdims=True))
    o_ref[...] = xb / jnp.maximum(n, 1e-12)


@jax.jit
def _norm1(x):
    blk = 640
    return pl.pallas_call(
        _norm1_body,
        out_shape=jax.ShapeDtypeStruct((_N_PAD, _D), jnp.float32),
        grid=(_N_PAD // blk,),
        in_specs=[pl.BlockSpec((blk, _D), lambda i: (i, 0))],
        out_specs=pl.BlockSpec((blk, _D), lambda i: (i, 0)),
    )(x)


def kernel(x, edge_index):
    row = edge_index[0].astype(jnp.int32)
    col = edge_index[1].astype(jnp.int32)
    npad = _EDGES_PAD - _N_EDGES
    rowp = jnp.concatenate([row, jnp.full((npad,), _N_NODES, jnp.int32)])
    colp = jnp.concatenate([col, jnp.zeros((npad,), jnp.int32)])
    row3 = rowp.reshape(_NW, _NCHUNK, _CHUNK)
    col3 = colp.reshape(_NW, _NCHUNK, _CHUNK)

    xpad = jnp.pad(x, ((0, _N_PAD - _N_NODES), (0, 0)))
    h = _norm1(xpad)
    outs = [h[:_N_NODES]]
    for _ in range(_NUM_HOPS):
        partials = _hop_sc(h, row3, col3)
        h = _norm2(partials)
        outs.append(h[:_N_NODES])
    return jnp.stack(outs)


# speedup vs baseline: 7.4017x; 1.5221x over previous
import jax
import jax.numpy as jnp
from jax import lax
from jax.experimental import pallas as pl
from jax.experimental.pallas import tpu as pltpu
from jax.experimental.pallas import tpu_sc as plsc

_N_NODES = 10000
_D = 128
_DH = 64
_N_EDGES = 320000
_NUM_HOPS = 3

_NSUB = 16
_CHUNK = 128
_JPB = 8
_BLKE = _CHUNK * _JPB
_NBLK = 20
_EPT = _BLKE * _NBLK
_EDGES_PAD = _NSUB * _EPT
_N_PAD = 10112
_RPS = _N_PAD // _NSUB
_STG = [(0, 128), (128, 128), (256, 128), (384, 128), (512, 120)]


def _hop_body(h_hbm, row_hbm, col_hbm, out_hbm, colv, rowv, g0, g1, table, acc,
              sem0, sem1, semz):
    c = lax.axis_index("c")
    s = lax.axis_index("s")
    rbase = s * _RPS
    gb = (g0, g1)
    gs = (sem0, sem1)

    for i, (off, nr) in enumerate(_STG):
        b = i % 2
        if i >= 2:
            po, pn = _STG[i - 2]
            pltpu.make_async_copy(gb[b].at[pl.ds(0, pn)],
                                  table.at[pl.ds(rbase + po, pn)], gs[b]).wait()
        pltpu.sync_copy(h_hbm.at[c, pl.ds(rbase + off, nr)], gb[b].at[pl.ds(0, nr)])
        pltpu.async_copy(gb[b].at[pl.ds(0, nr)], table.at[pl.ds(rbase + off, nr)],
                         gs[b])
    for i in (3, 4):
        off, nr = _STG[i]
        pltpu.make_async_copy(gb[i % 2].at[pl.ds(0, nr)],
                              table.at[pl.ds(rbase + off, nr)], gs[i % 2]).wait()

    zero = jnp.zeros((16,), jnp.float32)

    def _zrow(i, carry):
        for q in range(_DH // 16):
            g0[i, pl.ds(q * 16, 16)] = zero
        return carry

    lax.fori_loop(0, _CHUNK, _zrow, 0)
    for off, nr in _STG:
        pltpu.async_copy(g0.at[pl.ds(0, nr)], acc.at[pl.ds(rbase + off, nr)], semz)
    for off, nr in _STG:
        pltpu.make_async_copy(g0.at[pl.ds(0, nr)],
                              acc.at[pl.ds(rbase + off, nr)], semz).wait()
    plsc.subcore_barrier()

    def _cpidx(t, b):
        pltpu.sync_copy(col_hbm.at[s, pl.ds(t * _BLKE, _BLKE)], colv.at[b])
        pltpu.sync_copy(row_hbm.at[s, t], rowv.at[b])

    def _fire(tb, j, b):
        pltpu.async_copy(table.at[colv.at[tb, pl.ds(j * _CHUNK, _CHUNK)]],
                         gb[b], gs[b])

    def _gwait(tb, j, b):
        pltpu.make_async_copy(table.at[colv.at[tb, pl.ds(j * _CHUNK, _CHUNK)]],
                              gb[b], gs[b]).wait()

    _cpidx(0, 0)
    _fire(0, 0, 0)
    _fire(0, 1, 1)

    def _blk(t, carry):
        tb = lax.rem(t, 2)
        tb1 = lax.rem(t + 1, 2)

        @pl.when(t + 1 < _NBLK)
        def _():
            _cpidx(t + 1, tb1)

        for j in range(_JPB):
            b = j % 2
            _gwait(tb, j, b)
            pltpu.sync_copy(gb[b], acc.at[rowv.at[tb, j]], add=True)
            if j < _JPB - 2:
                _fire(tb, j + 2, b)
            else:
                @pl.when(t + 1 < _NBLK)
                def _():
                    _fire(tb1, j + 2 - _JPB, b)
        return carry

    lax.fori_loop(0, _NBLK, _blk, 0)
    plsc.subcore_barrier()

    for i, (off, nr) in enumerate(_STG):
        b = i % 2
        if i >= 2:
            po, pn = _STG[i - 2]
            pltpu.make_async_copy(gb[b].at[pl.ds(0, pn)],
                                  out_hbm.at[c, pl.ds(rbase + po, pn)], gs[b]).wait()
        pltpu.sync_copy(acc.at[pl.ds(rbase + off, nr)], gb[b].at[pl.ds(0, nr)])
        pltpu.async_copy(gb[b].at[pl.ds(0, nr)],
                         out_hbm.at[c, pl.ds(rbase + off, nr)], gs[b])
    for i in (3, 4):
        off, nr = _STG[i]
        pltpu.make_async_copy(gb[i % 2].at[pl.ds(0, nr)],
                              out_hbm.at[c, pl.ds(rbase + off, nr)], gs[i % 2]).wait()


@jax.jit
def _hop_sc(h, row4, col2):
    mesh = plsc.VectorSubcoreMesh(core_axis_name="c", subcore_axis_name="s")
    return pl.kernel(
        _hop_body,
        out_type=jax.ShapeDtypeStruct((2, _N_PAD, _DH), jnp.float32),
        mesh=mesh,
        compiler_params=pltpu.CompilerParams(use_tc_tiling_on_sc=False),
        scratch_types=[
            pltpu.VMEM((2, _BLKE), jnp.int32),
            pltpu.VMEM((2, _JPB, _CHUNK), jnp.int32),
            pltpu.VMEM((_CHUNK, _DH), jnp.float32),
            pltpu.VMEM((_CHUNK, _DH), jnp.float32),
            pltpu.VMEM_SHARED((_N_PAD, _DH), jnp.float32),
            pltpu.VMEM_SHARED((_N_PAD, _DH), jnp.float32),
            pltpu.SemaphoreType.DMA,
            pltpu.SemaphoreType.DMA,
            pltpu.SemaphoreType.DMA,
        ],
    )(h, row4, col2)


def _norm2_body(p_ref, o_ref, os_ref):
    su = jnp.concatenate([p_ref[0], p_ref[1]], axis=1)
    n = jnp.sqrt(jnp.sum(su * su, axis=1, keepdims=True))
    res = su / jnp.maximum(n, 1e-12)
    o_ref[...] = res
    os_ref[0] = res[:, :_DH]
    os_ref[1] = res[:, _DH:]


@jax.jit
def _norm2(p):
    blk = 632
    return pl.pallas_call(
        _norm2_body,
        out_shape=[
            jax.ShapeDtypeStruct((_N_PAD, _D), jnp.float32),
            jax.ShapeDtypeStruct((2, _N_PAD, _DH), jnp.float32),
        ],
        grid=(_N_PAD // blk,),
        in_specs=[pl.BlockSpec((2, blk, _DH), lambda i: (0, i, 0))],
        out_specs=[
            pl.BlockSpec((blk, _D), lambda i: (i, 0)),
            pl.BlockSpec((2, blk, _DH), lambda i: (0, i, 0)),
        ],
    )(p)


def _norm1_body(x_ref, o_ref, os_ref):
    xb = x_ref[...]
    n = jnp.sqrt(jnp.sum(xb * xb, axis=1, keepdims=True))
    res = xb / jnp.maximum(n, 1e-12)
    o_ref[...] = res
    os_ref[0] = res[:, :_DH]
    os_ref[1] = res[:, _DH:]


@jax.jit
def _norm1(x):
    blk = 632
    return pl.pallas_call(
        _norm1_body,
        out_shape=[
            jax.ShapeDtypeStruct((_N_PAD, _D), jnp.float32),
            jax.ShapeDtypeStruct((2, _N_PAD, _DH), jnp.float32),
        ],
        grid=(_N_PAD // blk,),
        in_specs=[pl.BlockSpec((blk, _D), lambda i: (i, 0))],
        out_specs=[
            pl.BlockSpec((blk, _D), lambda i: (i, 0)),
            pl.BlockSpec((2, blk, _DH), lambda i: (0, i, 0)),
        ],
    )(x)


def kernel(x, edge_index):
    row = edge_index[0].astype(jnp.int32)
    col = edge_index[1].astype(jnp.int32)
    npad = _EDGES_PAD - _N_EDGES
    rowp = jnp.concatenate([row, jnp.full((npad,), _N_NODES, jnp.int32)])
    colp = jnp.concatenate([col, jnp.zeros((npad,), jnp.int32)])
    row4 = rowp.reshape(_NSUB, _NBLK, _JPB, _CHUNK)
    col2 = colp.reshape(_NSUB, _EPT)

    xpad = jnp.pad(x, ((0, _N_PAD - _N_NODES), (0, 0)))
    h, hs = _norm1(xpad)
    outs = [h[:_N_NODES]]
    for _ in range(_NUM_HOPS):
        partials = _hop_sc(hs, row4, col2)
        h, hs = _norm2(partials)
        outs.append(h[:_N_NODES])
    return jnp.stack(outs)


# speedup vs baseline: 8.6189x; 1.1644x over previous
import jax
import jax.numpy as jnp
from jax import lax
from jax.experimental import pallas as pl
from jax.experimental.pallas import tpu as pltpu
from jax.experimental.pallas import tpu_sc as plsc

_N_NODES = 10000
_D = 128
_DH = 64
_N_EDGES = 320000
_NUM_HOPS = 3

_NSUB = 16
_CHUNK = 128
_JPB = 8
_BLKE = _CHUNK * _JPB
_NBLK = 20
_EPT = _BLKE * _NBLK
_EDGES_PAD = _NSUB * _EPT
_N_PAD = 10112
_RPS = _N_PAD // _NSUB
_STG = [(0, 128), (128, 128), (256, 128), (384, 128), (512, 120)]


def _body(x_hbm, row_hbm, col_hbm, out_hbm, sq_hbm, colv, rowv, g0, g1, g2, g3,
          gz, sqv, sqo, table, acc, gsem0, gsem1, gsem2, gsem3,
          ssem0, ssem1, ssem2, ssem3, semz, bsem):
    c = lax.axis_index("c")
    s = lax.axis_index("s")
    rbase = s * _RPS
    gb = (g0, g1, g2, g3)
    gs = (gsem0, gsem1, gsem2, gsem3)
    ss = (ssem0, ssem1, ssem2, ssem3)
    zero = jnp.zeros((16,), jnp.float32)

    def _zrow(i, carry):
        for q in range(_DH // 16):
            gz[i, pl.ds(q * 16, 16)] = zero
        return carry

    lax.fori_loop(0, _CHUNK, _zrow, 0)

    for i, (off, nr) in enumerate(_STG):
        b = i % 2
        if i >= 2:
            po, pn = _STG[i - 2]
            pltpu.make_async_copy(gb[b].at[pl.ds(0, pn)],
                                  acc.at[pl.ds(rbase + po, pn)], gs[b]).wait()
        pltpu.sync_copy(x_hbm.at[c, pl.ds(rbase + off, nr)], gb[b].at[pl.ds(0, nr)])
        pltpu.async_copy(gb[b].at[pl.ds(0, nr)], acc.at[pl.ds(rbase + off, nr)],
                         gs[b])
    for i in (3, 4):
        off, nr = _STG[i]
        pltpu.make_async_copy(gb[i % 2].at[pl.ds(0, nr)],
                              acc.at[pl.ds(rbase + off, nr)], gs[i % 2]).wait()
    plsc.subcore_barrier()

    def _normalize(k):
        lane = lax.iota(jnp.int32, 16)
        for off, nr in _STG:
            pltpu.sync_copy(acc.at[pl.ds(rbase + off, nr)], g0.at[pl.ds(0, nr)])

            def _sumsq(g16, carry, _off=off):
                ridx = g16 * 16 + lane
                t = jnp.zeros((16,), jnp.float32)
                for q in range(_DH):
                    v = plsc.load_gather(g0, [ridx, jnp.full((16,), q, jnp.int32)])
                    t = t + v * v
                sqv[pl.ds(_off + g16 * 16, 16)] = t
                return carry

            lax.fori_loop(0, _CHUNK // 16, _sumsq, 0)
        pltpu.sync_copy(sqv.at[pl.ds(0, _RPS)], sq_hbm.at[c, pl.ds(rbase, _RPS)])
        plsc.subcore_barrier()
        pltpu.core_barrier(bsem, core_axis_name="c")
        pltpu.sync_copy(sq_hbm.at[1 - c, pl.ds(rbase, _RPS)],
                        sqo.at[pl.ds(0, _RPS)])

        def _rsqrt(i, carry):
            t = sqv[pl.ds(i * 16, 16)] + sqo[pl.ds(i * 16, 16)]
            bits = plsc.bitcast(t, jnp.int32)
            bits = 0x5F3759DF - lax.shift_right_logical(bits, 1)
            y = plsc.bitcast(bits, jnp.float32)
            for _ in range(3):
                y = y * (1.5 - 0.5 * t * y * y)
            sqv[pl.ds(i * 16, 16)] = jnp.where(t > 0.0, y, 0.0)
            return carry

        lax.fori_loop(0, _RPS // 16 + 1, _rsqrt, 0)

        notlast = k < _NUM_HOPS
        for i, (off, nr) in enumerate(_STG):
            b = i % 2
            if i >= 2:
                po, pn = _STG[i - 2]
                pltpu.make_async_copy(
                    gb[b].at[pl.ds(0, pn)],
                    out_hbm.at[k, c, pl.ds(rbase + po, pn)], gs[b]).wait()

                @pl.when(notlast)
                def _(_b=b, _po=po, _pn=pn):
                    pltpu.make_async_copy(
                        gb[_b].at[pl.ds(0, _pn)],
                        table.at[pl.ds(rbase + _po, _pn)], ss[_b]).wait()
            pltpu.sync_copy(acc.at[pl.ds(rbase + off, nr)], gb[b].at[pl.ds(0, nr)])

            @pl.when(notlast)
            def _(_off=off, _nr=nr):
                pltpu.async_copy(gz.at[pl.ds(0, _nr)],
                                 acc.at[pl.ds(rbase + _off, _nr)], semz)

            def _scale(g16, carry, _b=b, _off=off):
                f16 = sqv[pl.ds(_off + g16 * 16, 16)]
                for rr in range(16):
                    r = g16 * 16 + rr
                    f = f16[rr]
                    for q in range(_DH // 16):
                        gb[_b][r, pl.ds(q * 16, 16)] = (
                            gb[_b][r, pl.ds(q * 16, 16)] * f)
                return carry

            lax.fori_loop(0, _CHUNK // 16, _scale, 0)
            pltpu.async_copy(gb[b].at[pl.ds(0, nr)],
                             out_hbm.at[k, c, pl.ds(rbase + off, nr)], gs[b])

            @pl.when(notlast)
            def _(_b=b, _off=off, _nr=nr):
                pltpu.async_copy(gb[_b].at[pl.ds(0, _nr)],
                                 table.at[pl.ds(rbase + _off, _nr)], ss[_b])
        for i in (3, 4):
            off, nr = _STG[i]
            pltpu.make_async_copy(gb[i % 2].at[pl.ds(0, nr)],
                                  out_hbm.at[k, c, pl.ds(rbase + off, nr)],
                                  gs[i % 2]).wait()

            @pl.when(notlast)
            def _(_i=i, _off=off, _nr=nr):
                pltpu.make_async_copy(gb[_i % 2].at[pl.ds(0, _nr)],
                                      table.at[pl.ds(rbase + _off, _nr)],
                                      ss[_i % 2]).wait()

        @pl.when(notlast)
        def _():
            for off, nr in _STG:
                pltpu.make_async_copy(gz.at[pl.ds(0, nr)],
                                      acc.at[pl.ds(rbase + off, nr)], semz).wait()
        plsc.subcore_barrier()

    def _edge_loop():
        def _cpidx(t, b):
            pltpu.sync_copy(col_hbm.at[s, pl.ds(t * _BLKE, _BLKE)], colv.at[b])
            pltpu.sync_copy(row_hbm.at[s, t], rowv.at[b])

        def _fire_g(tb, j, b):
            pltpu.async_copy(table.at[colv.at[tb, pl.ds(j * _CHUNK, _CHUNK)]],
                             gb[b], gs[b])

        def _wait_g(tb, j, b):
            pltpu.make_async_copy(
                table.at[colv.at[tb, pl.ds(j * _CHUNK, _CHUNK)]],
                gb[b], gs[b]).wait()

        def _fire_s(tb, j, b):
            pltpu.async_copy(gb[b], acc.at[rowv.at[tb, j]], ss[b], add=True)

        def _wait_s(tb, j, b):
            pltpu.make_async_copy(gb[b], acc.at[rowv.at[tb, j]], ss[b]).wait()

        _cpidx(0, 0)
        _fire_g(0, 0, 0)
        _fire_g(0, 1, 1)

        def _blk(t, carry):
            tb = lax.rem(t, 2)
            tb1 = lax.rem(t + 1, 2)

            @pl.when(t + 1 < _NBLK)
            def _():
                _cpidx(t + 1, tb1)

            for j in range(_JPB):
                b = j % 4
                b2 = (j + 2) % 4
                _wait_g(tb, j, b)
                _fire_s(tb, j, b)
                if j < 2:
                    @pl.when(t > 0)
                    def _():
                        _wait_s(tb1, j + _JPB - 2, b2)
                else:
                    _wait_s(tb, j - 2, b2)
                if j < _JPB - 2:
                    _fire_g(tb, j + 2, b2)
                else:
                    @pl.when(t + 1 < _NBLK)
                    def _():
                        _fire_g(tb1, j + 2 - _JPB, b2)
            return carry

        lax.fori_loop(0, _NBLK, _blk, 0)
        _wait_s((_NBLK - 1) % 2, _JPB - 2, (_JPB - 2) % 4)
        _wait_s((_NBLK - 1) % 2, _JPB - 1, (_JPB - 1) % 4)
        plsc.subcore_barrier()

    def _level(k, carry):
        @pl.when(k > 0)
        def _():
            _edge_loop()

        _normalize(k)
        return carry

    lax.fori_loop(0, _NUM_HOPS + 1, _level, 0)


@jax.jit
def _fused_sc(xs, row4, col2):
    mesh = plsc.VectorSubcoreMesh(core_axis_name="c", subcore_axis_name="s")
    return pl.kernel(
        _body,
        out_type=(
            jax.ShapeDtypeStruct((_NUM_HOPS + 1, 2, _N_PAD, _DH), jnp.float32),
            jax.ShapeDtypeStruct((2, _N_PAD), jnp.float32),
        ),
        mesh=mesh,
        compiler_params=pltpu.CompilerParams(use_tc_tiling_on_sc=False,
                                             needs_layout_passes=False),
        scratch_types=[
            pltpu.VMEM((2, _BLKE), jnp.int32),
            pltpu.VMEM((2, _JPB, _CHUNK), jnp.int32),
            pltpu.VMEM((_CHUNK, _DH), jnp.float32),
            pltpu.VMEM((_CHUNK, _DH), jnp.float32),
            pltpu.VMEM((_CHUNK, _DH), jnp.float32),
            pltpu.VMEM((_CHUNK, _DH), jnp.float32),
            pltpu.VMEM((_CHUNK, _DH), jnp.float32),
            pltpu.VMEM((_RPS + 8,), jnp.float32),
            pltpu.VMEM((_RPS + 8,), jnp.float32),
            pltpu.VMEM_SHARED((_N_PAD, _DH), jnp.float32),
            pltpu.VMEM_SHARED((_N_PAD, _DH), jnp.float32),
            pltpu.SemaphoreType.DMA,
            pltpu.SemaphoreType.DMA,
            pltpu.SemaphoreType.DMA,
            pltpu.SemaphoreType.DMA,
            pltpu.SemaphoreType.DMA,
            pltpu.SemaphoreType.DMA,
            pltpu.SemaphoreType.DMA,
            pltpu.SemaphoreType.DMA,
            pltpu.SemaphoreType.DMA,
            pltpu.SemaphoreType.REGULAR,
        ],
    )(xs, row4, col2)


def kernel(x, edge_index):
    row = edge_index[0].astype(jnp.int32)
    col = edge_index[1].astype(jnp.int32)
    npad = _EDGES_PAD - _N_EDGES
    rowp = jnp.concatenate([row, jnp.full((npad,), _N_NODES, jnp.int32)])
    colp = jnp.concatenate([col, jnp.zeros((npad,), jnp.int32)])
    row4 = rowp.reshape(_NSUB, _NBLK, _JPB, _CHUNK)
    col2 = colp.reshape(_NSUB, _EPT)

    xpad = jnp.pad(x, ((0, _N_PAD - _N_NODES), (0, 0)))
    xs = jnp.stack([xpad[:, :_DH], xpad[:, _DH:]])
    out, _ = _fused_sc(xs, row4, col2)
    full = jnp.concatenate([out[:, 0], out[:, 1]], axis=-1)
    return full[:, :_N_NODES]


# speedup vs baseline: 9.0590x; 1.0511x over previous
import jax
import jax.numpy as jnp
from jax import lax
from jax.experimental import pallas as pl
from jax.experimental.pallas import tpu as pltpu
from jax.experimental.pallas import tpu_sc as plsc

_N_NODES = 10000
_D = 128
_DH = 64
_N_EDGES = 320000
_NUM_HOPS = 3

_NSUB = 16
_CHUNK = 128
_JPB = 8
_BLKE = _CHUNK * _JPB
_NBLK = 20
_EPT = _BLKE * _NBLK
_EDGES_PAD = _NSUB * _EPT
_N_PAD = 10112
_RPS = _N_PAD // _NSUB
_STG = [(0, 128), (128, 128), (256, 128), (384, 128), (512, 120)]


def _body(x_hbm, row_hbm, col_hbm, out_hbm, sq_hbm, colv, rowv, g0, g1, g2, g3,
          gz, sqv, sqo, table, acc, gsem0, gsem1, gsem2, gsem3,
          ssem0, ssem1, ssem2, ssem3, semz, bsem):
    c = lax.axis_index("c")
    s = lax.axis_index("s")
    rbase = s * _RPS
    gb = (g0, g1, g2, g3)
    gs = (gsem0, gsem1, gsem2, gsem3)
    ss = (ssem0, ssem1, ssem2, ssem3)
    zero = jnp.zeros((16,), jnp.float32)

    def _zrow(i, carry):
        for q in range(_DH // 16):
            gz[i, pl.ds(q * 16, 16)] = zero
        return carry

    lax.fori_loop(0, _CHUNK, _zrow, 0)

    for i, (off, nr) in enumerate(_STG):
        b = i % 2
        if i >= 2:
            po, pn = _STG[i - 2]
            pltpu.make_async_copy(gb[b].at[pl.ds(0, pn)],
                                  acc.at[pl.ds(rbase + po, pn)], gs[b]).wait()
        pltpu.sync_copy(x_hbm.at[c, pl.ds(rbase + off, nr)], gb[b].at[pl.ds(0, nr)])
        pltpu.async_copy(gb[b].at[pl.ds(0, nr)], acc.at[pl.ds(rbase + off, nr)],
                         gs[b])
    for i in (3, 4):
        off, nr = _STG[i]
        pltpu.make_async_copy(gb[i % 2].at[pl.ds(0, nr)],
                              acc.at[pl.ds(rbase + off, nr)], gs[i % 2]).wait()
    plsc.subcore_barrier()

    def _normalize(k):
        lane = lax.iota(jnp.int32, 16)
        for off, nr in _STG:
            pltpu.sync_copy(acc.at[pl.ds(rbase + off, nr)], g0.at[pl.ds(0, nr)])

            def _sumsq(g16, carry, _off=off):
                ridx = g16 * 16 + lane
                t = [jnp.zeros((16,), jnp.float32) for _ in range(4)]
                for q in range(_DH):
                    v = plsc.load_gather(g0, [ridx, jnp.full((16,), q, jnp.int32)])
                    t[q % 4] = t[q % 4] + v * v
                sqv[pl.ds(_off + g16 * 16, 16)] = (t[0] + t[1]) + (t[2] + t[3])
                return carry

            lax.fori_loop(0, _CHUNK // 16, _sumsq, 0)
        pltpu.sync_copy(sqv.at[pl.ds(0, _RPS)], sq_hbm.at[c, pl.ds(rbase, _RPS)])
        pltpu.core_barrier(bsem, core_axis_name="c")
        pltpu.sync_copy(sq_hbm.at[1 - c, pl.ds(rbase, _RPS)],
                        sqo.at[pl.ds(0, _RPS)])

        def _rsqrt(i, carry):
            t = sqv[pl.ds(i * 16, 16)] + sqo[pl.ds(i * 16, 16)]
            bits = plsc.bitcast(t, jnp.int32)
            bits = 0x5F3759DF - lax.shift_right_logical(bits, 1)
            y = plsc.bitcast(bits, jnp.float32)
            for _ in range(3):
                y = y * (1.5 - 0.5 * t * y * y)
            sqv[pl.ds(i * 16, 16)] = jnp.where(t > 0.0, y, 0.0)
            return carry

        lax.fori_loop(0, _RPS // 16 + 1, _rsqrt, 0)

        notlast = k < _NUM_HOPS
        for i, (off, nr) in enumerate(_STG):
            b = i % 2
            if i >= 2:
                po, pn = _STG[i - 2]
                pltpu.make_async_copy(
                    gb[b].at[pl.ds(0, pn)],
                    out_hbm.at[k, c, pl.ds(rbase + po, pn)], gs[b]).wait()

                @pl.when(notlast)
                def _(_b=b, _po=po, _pn=pn):
                    pltpu.make_async_copy(
                        gb[_b].at[pl.ds(0, _pn)],
                        table.at[pl.ds(rbase + _po, _pn)], ss[_b]).wait()
            pltpu.sync_copy(acc.at[pl.ds(rbase + off, nr)], gb[b].at[pl.ds(0, nr)])

            @pl.when(notlast)
            def _(_off=off, _nr=nr):
                pltpu.async_copy(gz.at[pl.ds(0, _nr)],
                                 acc.at[pl.ds(rbase + _off, _nr)], semz)

            def _scale(g16, carry, _b=b, _off=off):
                f16 = sqv[pl.ds(_off + g16 * 16, 16)]
                for rr in range(16):
                    r = g16 * 16 + rr
                    f = f16[rr]
                    for q in range(_DH // 16):
                        gb[_b][r, pl.ds(q * 16, 16)] = (
                            gb[_b][r, pl.ds(q * 16, 16)] * f)
                return carry

            lax.fori_loop(0, _CHUNK // 16, _scale, 0)
            pltpu.async_copy(gb[b].at[pl.ds(0, nr)],
                             out_hbm.at[k, c, pl.ds(rbase + off, nr)], gs[b])

            @pl.when(notlast)
            def _(_b=b, _off=off, _nr=nr):
                pltpu.async_copy(gb[_b].at[pl.ds(0, _nr)],
                                 table.at[pl.ds(rbase + _off, _nr)], ss[_b])
        for i in (3, 4):
            off, nr = _STG[i]
            pltpu.make_async_copy(gb[i % 2].at[pl.ds(0, nr)],
                                  out_hbm.at[k, c, pl.ds(rbase + off, nr)],
                                  gs[i % 2]).wait()

            @pl.when(notlast)
            def _(_i=i, _off=off, _nr=nr):
                pltpu.make_async_copy(gb[_i % 2].at[pl.ds(0, _nr)],
                                      table.at[pl.ds(rbase + _off, _nr)],
                                      ss[_i % 2]).wait()

        @pl.when(notlast)
        def _():
            for off, nr in _STG:
                pltpu.make_async_copy(gz.at[pl.ds(0, nr)],
                                      acc.at[pl.ds(rbase + off, nr)], semz).wait()
        plsc.subcore_barrier()

    def _edge_loop():
        def _cpidx(t, b):
            pltpu.sync_copy(col_hbm.at[s, pl.ds(t * _BLKE, _BLKE)], colv.at[b])
            pltpu.sync_copy(row_hbm.at[s, t], rowv.at[b])

        def _fire_g(tb, j, b):
            pltpu.async_copy(table.at[colv.at[tb, pl.ds(j * _CHUNK, _CHUNK)]],
                             gb[b], gs[b])

        def _wait_g(tb, j, b):
            pltpu.make_async_copy(
                table.at[colv.at[tb, pl.ds(j * _CHUNK, _CHUNK)]],
                gb[b], gs[b]).wait()

        def _fire_s(tb, j, b):
            pltpu.async_copy(gb[b], acc.at[rowv.at[tb, j]], ss[b], add=True)

        def _wait_s(tb, j, b):
            pltpu.make_async_copy(gb[b], acc.at[rowv.at[tb, j]], ss[b]).wait()

        _cpidx(0, 0)
        _fire_g(0, 0, 0)
        _fire_g(0, 1, 1)

        def _blk(t, carry):
            tb = lax.rem(t, 2)
            tb1 = lax.rem(t + 1, 2)

            @pl.when(t + 1 < _NBLK)
            def _():
                _cpidx(t + 1, tb1)

            for j in range(_JPB):
                b = j % 4
                b2 = (j + 2) % 4
                _wait_g(tb, j, b)
                _fire_s(tb, j, b)
                if j < 2:
                    @pl.when(t > 0)
                    def _():
                        _wait_s(tb1, j + _JPB - 2, b2)
                else:
                    _wait_s(tb, j - 2, b2)
                if j < _JPB - 2:
                    _fire_g(tb, j + 2, b2)
                else:
                    @pl.when(t + 1 < _NBLK)
                    def _():
                        _fire_g(tb1, j + 2 - _JPB, b2)
            return carry

        lax.fori_loop(0, _NBLK, _blk, 0)
        _wait_s((_NBLK - 1) % 2, _JPB - 2, (_JPB - 2) % 4)
        _wait_s((_NBLK - 1) % 2, _JPB - 1, (_JPB - 1) % 4)
        plsc.subcore_barrier()

    def _level(k, carry):
        @pl.when(k > 0)
        def _():
            _edge_loop()

        _normalize(k)
        return carry

    lax.fori_loop(0, _NUM_HOPS + 1, _level, 0)


@jax.jit
def _fused_sc(xs, row4, col2):
    mesh = plsc.VectorSubcoreMesh(core_axis_name="c", subcore_axis_name="s")
    return pl.kernel(
        _body,
        out_type=(
            jax.ShapeDtypeStruct((_NUM_HOPS + 1, 2, _N_PAD, _DH), jnp.float32),
            jax.ShapeDtypeStruct((2, _N_PAD), jnp.float32),
        ),
        mesh=mesh,
        compiler_params=pltpu.CompilerParams(use_tc_tiling_on_sc=False,
                                             needs_layout_passes=False),
        scratch_types=[
            pltpu.VMEM((2, _BLKE), jnp.int32),
            pltpu.VMEM((2, _JPB, _CHUNK), jnp.int32),
            pltpu.VMEM((_CHUNK, _DH), jnp.float32),
            pltpu.VMEM((_CHUNK, _DH), jnp.float32),
            pltpu.VMEM((_CHUNK, _DH), jnp.float32),
            pltpu.VMEM((_CHUNK, _DH), jnp.float32),
            pltpu.VMEM((_CHUNK, _DH), jnp.float32),
            pltpu.VMEM((_RPS + 8,), jnp.float32),
            pltpu.VMEM((_RPS + 8,), jnp.float32),
            pltpu.VMEM_SHARED((_N_PAD, _DH), jnp.float32),
            pltpu.VMEM_SHARED((_N_PAD, _DH), jnp.float32),
            pltpu.SemaphoreType.DMA,
            pltpu.SemaphoreType.DMA,
            pltpu.SemaphoreType.DMA,
            pltpu.SemaphoreType.DMA,
            pltpu.SemaphoreType.DMA,
            pltpu.SemaphoreType.DMA,
            pltpu.SemaphoreType.DMA,
            pltpu.SemaphoreType.DMA,
            pltpu.SemaphoreType.DMA,
            pltpu.SemaphoreType.REGULAR,
        ],
    )(xs, row4, col2)


def kernel(x, edge_index):
    row = edge_index[0].astype(jnp.int32)
    col = edge_index[1].astype(jnp.int32)
    npad = _EDGES_PAD - _N_EDGES
    rowp = jnp.concatenate([row, jnp.full((npad,), _N_NODES, jnp.int32)])
    colp = jnp.concatenate([col, jnp.zeros((npad,), jnp.int32)])
    row4 = rowp.reshape(_NSUB, _NBLK, _JPB, _CHUNK)
    col2 = colp.reshape(_NSUB, _EPT)

    xpad = jnp.pad(x, ((0, _N_PAD - _N_NODES), (0, 0)))
    xs = jnp.stack([xpad[:, :_DH], xpad[:, _DH:]])
    out, _ = _fused_sc(xs, row4, col2)
    full = jnp.concatenate([out[:, 0], out[:, 1]], axis=-1)
    return full[:, :_N_NODES]


# speedup vs baseline: 9.1647x; 1.0117x over previous
import jax
import jax.numpy as jnp
from jax import lax
from jax.experimental import pallas as pl
from jax.experimental.pallas import tpu as pltpu
from jax.experimental.pallas import tpu_sc as plsc

_N_NODES = 10000
_D = 128
_DH = 64
_N_EDGES = 320000
_NUM_HOPS = 3

_NSUB = 16
_CHUNK = 128
_JPB = 8
_BLKE = _CHUNK * _JPB
_NBLK = 20
_EPT = _BLKE * _NBLK
_EDGES_PAD = _NSUB * _EPT
_N_PAD = 10112
_RPS = _N_PAD // _NSUB
_STG = [(0, 128), (128, 128), (256, 128), (384, 128), (512, 120)]


def _body(x_hbm, row_hbm, col_hbm, out_hbm, sq_hbm, colv, rowv, g0, g1, g2, g3,
          gz, sqv, sqo, table, acc, gsem0, gsem1, gsem2, gsem3,
          ssem0, ssem1, ssem2, ssem3, semz, bsem):
    c = lax.axis_index("c")
    s = lax.axis_index("s")
    rbase = s * _RPS
    gb = (g0, g1, g2, g3)
    gs = (gsem0, gsem1, gsem2, gsem3)
    ss = (ssem0, ssem1, ssem2, ssem3)
    zero = jnp.zeros((16,), jnp.float32)

    def _zrow(i, carry):
        for q in range(_DH // 16):
            gz[i, pl.ds(q * 16, 16)] = zero
        return carry

    lax.fori_loop(0, _CHUNK, _zrow, 0)

    for i, (off, nr) in enumerate(_STG):
        b = i % 2
        if i >= 2:
            po, pn = _STG[i - 2]
            pltpu.make_async_copy(gb[b].at[pl.ds(0, pn)],
                                  acc.at[pl.ds(rbase + po, pn)], gs[b]).wait()
        pltpu.sync_copy(x_hbm.at[c, pl.ds(rbase + off, nr)], gb[b].at[pl.ds(0, nr)])
        pltpu.async_copy(gb[b].at[pl.ds(0, nr)], acc.at[pl.ds(rbase + off, nr)],
                         gs[b])
    for i in (3, 4):
        off, nr = _STG[i]
        pltpu.make_async_copy(gb[i % 2].at[pl.ds(0, nr)],
                              acc.at[pl.ds(rbase + off, nr)], gs[i % 2]).wait()
    plsc.subcore_barrier()

    def _normalize(k):
        lane = lax.iota(jnp.int32, 16)
        pltpu.async_copy(acc.at[pl.ds(rbase + _STG[0][0], _STG[0][1])],
                         g0.at[pl.ds(0, _STG[0][1])], gs[0])
        for i, (off, nr) in enumerate(_STG):
            b = i % 2
            pltpu.make_async_copy(acc.at[pl.ds(rbase + off, nr)],
                                  gb[b].at[pl.ds(0, nr)], gs[b]).wait()
            if i + 1 < len(_STG):
                no, nn = _STG[i + 1]
                pltpu.async_copy(acc.at[pl.ds(rbase + no, nn)],
                                 gb[1 - b].at[pl.ds(0, nn)], gs[1 - b])

            def _sumsq(g16, carry, _off=off, _g=gb[b]):
                ridx = g16 * 16 + lane
                t = [jnp.zeros((16,), jnp.float32) for _ in range(4)]
                for q in range(_DH):
                    v = plsc.load_gather(_g, [ridx, jnp.full((16,), q, jnp.int32)])
                    t[q % 4] = t[q % 4] + v * v
                sqv[pl.ds(_off + g16 * 16, 16)] = (t[0] + t[1]) + (t[2] + t[3])
                return carry

            lax.fori_loop(0, _CHUNK // 16, _sumsq, 0)
        pltpu.sync_copy(sqv.at[pl.ds(0, _RPS)], sq_hbm.at[c, pl.ds(rbase, _RPS)])
        pltpu.core_barrier(bsem, core_axis_name="c")
        pltpu.sync_copy(sq_hbm.at[1 - c, pl.ds(rbase, _RPS)],
                        sqo.at[pl.ds(0, _RPS)])

        def _rsqrt(i, carry):
            t = sqv[pl.ds(i * 16, 16)] + sqo[pl.ds(i * 16, 16)]
            bits = plsc.bitcast(t, jnp.int32)
            bits = 0x5F3759DF - lax.shift_right_logical(bits, 1)
            y = plsc.bitcast(bits, jnp.float32)
            for _ in range(3):
                y = y * (1.5 - 0.5 * t * y * y)
            sqv[pl.ds(i * 16, 16)] = jnp.where(t > 0.0, y, 0.0)
            return carry

        lax.fori_loop(0, _RPS // 16 + 1, _rsqrt, 0)

        notlast = k < _NUM_HOPS
        for i, (off, nr) in enumerate(_STG):
            b = i % 2
            if i >= 2:
                po, pn = _STG[i - 2]
                pltpu.make_async_copy(
                    gb[b].at[pl.ds(0, pn)],
                    out_hbm.at[k, c, pl.ds(rbase + po, pn)], gs[b]).wait()

                @pl.when(notlast)
                def _(_b=b, _po=po, _pn=pn):
                    pltpu.make_async_copy(
                        gb[_b].at[pl.ds(0, _pn)],
                        table.at[pl.ds(rbase + _po, _pn)], ss[_b]).wait()
            pltpu.sync_copy(acc.at[pl.ds(rbase + off, nr)], gb[b].at[pl.ds(0, nr)])

            @pl.when(notlast)
            def _(_off=off, _nr=nr):
                pltpu.async_copy(gz.at[pl.ds(0, _nr)],
                                 acc.at[pl.ds(rbase + _off, _nr)], semz)

            def _scale(g16, carry, _b=b, _off=off):
                f16 = sqv[pl.ds(_off + g16 * 16, 16)]
                for rr in range(16):
                    r = g16 * 16 + rr
                    f = f16[rr]
                    for q in range(_DH // 16):
                        gb[_b][r, pl.ds(q * 16, 16)] = (
                            gb[_b][r, pl.ds(q * 16, 16)] * f)
                return carry

            lax.fori_loop(0, _CHUNK // 16, _scale, 0)
            pltpu.async_copy(gb[b].at[pl.ds(0, nr)],
                             out_hbm.at[k, c, pl.ds(rbase + off, nr)], gs[b])

            @pl.when(notlast)
            def _(_b=b, _off=off, _nr=nr):
                pltpu.async_copy(gb[_b].at[pl.ds(0, _nr)],
                                 table.at[pl.ds(rbase + _off, _nr)], ss[_b])
        for i in (3, 4):
            off, nr = _STG[i]
            pltpu.make_async_copy(gb[i % 2].at[pl.ds(0, nr)],
                                  out_hbm.at[k, c, pl.ds(rbase + off, nr)],
                                  gs[i % 2]).wait()

            @pl.when(notlast)
            def _(_i=i, _off=off, _nr=nr):
                pltpu.make_async_copy(gb[_i % 2].at[pl.ds(0, _nr)],
                                      table.at[pl.ds(rbase + _off, _nr)],
                                      ss[_i % 2]).wait()

        @pl.when(notlast)
        def _():
            for off, nr in _STG:
                pltpu.make_async_copy(gz.at[pl.ds(0, nr)],
                                      acc.at[pl.ds(rbase + off, nr)], semz).wait()
        plsc.subcore_barrier()

    def _edge_loop():
        def _cpidx(t, b):
            pltpu.sync_copy(col_hbm.at[s, pl.ds(t * _BLKE, _BLKE)], colv.at[b])
            pltpu.sync_copy(row_hbm.at[s, t], rowv.at[b])

        def _fire_g(tb, j, b):
            pltpu.async_copy(table.at[colv.at[tb, pl.ds(j * _CHUNK, _CHUNK)]],
                             gb[b], gs[b])

        def _wait_g(tb, j, b):
            pltpu.make_async_copy(
                table.at[colv.at[tb, pl.ds(j * _CHUNK, _CHUNK)]],
                gb[b], gs[b]).wait()

        def _fire_s(tb, j, b):
            pltpu.async_copy(gb[b], acc.at[rowv.at[tb, j]], ss[b], add=True)

        def _wait_s(tb, j, b):
            pltpu.make_async_copy(gb[b], acc.at[rowv.at[tb, j]], ss[b]).wait()

        _cpidx(0, 0)
        _fire_g(0, 0, 0)
        _fire_g(0, 1, 1)

        def _blk(t, carry):
            tb = lax.rem(t, 2)
            tb1 = lax.rem(t + 1, 2)

            @pl.when(t + 1 < _NBLK)
            def _():
                _cpidx(t + 1, tb1)

            for j in range(_JPB):
                b = j % 4
                b2 = (j + 2) % 4
                _wait_g(tb, j, b)
                _fire_s(tb, j, b)
                if j < 2:
                    @pl.when(t > 0)
                    def _():
                        _wait_s(tb1, j + _JPB - 2, b2)
                else:
                    _wait_s(tb, j - 2, b2)
                if j < _JPB - 2:
                    _fire_g(tb, j + 2, b2)
                else:
                    @pl.when(t + 1 < _NBLK)
                    def _():
                        _fire_g(tb1, j + 2 - _JPB, b2)
            return carry

        lax.fori_loop(0, _NBLK, _blk, 0)
        _wait_s((_NBLK - 1) % 2, _JPB - 2, (_JPB - 2) % 4)
        _wait_s((_NBLK - 1) % 2, _JPB - 1, (_JPB - 1) % 4)
        plsc.subcore_barrier()

    def _level(k, carry):
        @pl.when(k > 0)
        def _():
            _edge_loop()

        _normalize(k)
        return carry

    lax.fori_loop(0, _NUM_HOPS + 1, _level, 0)


@jax.jit
def _fused_sc(xs, row4, col2):
    mesh = plsc.VectorSubcoreMesh(core_axis_name="c", subcore_axis_name="s")
    return pl.kernel(
        _body,
        out_type=(
            jax.ShapeDtypeStruct((_NUM_HOPS + 1, 2, _N_PAD, _DH), jnp.float32),
            jax.ShapeDtypeStruct((2, _N_PAD), jnp.float32),
        ),
        mesh=mesh,
        compiler_params=pltpu.CompilerParams(use_tc_tiling_on_sc=False,
                                             needs_layout_passes=False),
        scratch_types=[
            pltpu.VMEM((2, _BLKE), jnp.int32),
            pltpu.VMEM((2, _JPB, _CHUNK), jnp.int32),
            pltpu.VMEM((_CHUNK, _DH), jnp.float32),
            pltpu.VMEM((_CHUNK, _DH), jnp.float32),
            pltpu.VMEM((_CHUNK, _DH), jnp.float32),
            pltpu.VMEM((_CHUNK, _DH), jnp.float32),
            pltpu.VMEM((_CHUNK, _DH), jnp.float32),
            pltpu.VMEM((_RPS + 8,), jnp.float32),
            pltpu.VMEM((_RPS + 8,), jnp.float32),
            pltpu.VMEM_SHARED((_N_PAD, _DH), jnp.float32),
            pltpu.VMEM_SHARED((_N_PAD, _DH), jnp.float32),
            pltpu.SemaphoreType.DMA,
            pltpu.SemaphoreType.DMA,
            pltpu.SemaphoreType.DMA,
            pltpu.SemaphoreType.DMA,
            pltpu.SemaphoreType.DMA,
            pltpu.SemaphoreType.DMA,
            pltpu.SemaphoreType.DMA,
            pltpu.SemaphoreType.DMA,
            pltpu.SemaphoreType.DMA,
            pltpu.SemaphoreType.REGULAR,
        ],
    )(xs, row4, col2)


def kernel(x, edge_index):
    row = edge_index[0].astype(jnp.int32)
    col = edge_index[1].astype(jnp.int32)
    npad = _EDGES_PAD - _N_EDGES
    rowp = jnp.concatenate([row, jnp.full((npad,), _N_NODES, jnp.int32)])
    colp = jnp.concatenate([col, jnp.zeros((npad,), jnp.int32)])
    row4 = rowp.reshape(_NSUB, _NBLK, _JPB, _CHUNK)
    col2 = colp.reshape(_NSUB, _EPT)

    xpad = jnp.pad(x, ((0, _N_PAD - _N_NODES), (0, 0)))
    xs = jnp.stack([xpad[:, :_DH], xpad[:, _DH:]])
    out, _ = _fused_sc(xs, row4, col2)
    full = jnp.concatenate([out[:, 0], out[:, 1]], axis=-1)
    return full[:, :_N_NODES]
